# Initial kernel scaffold; baseline (speedup 1.0000x reference)
import jax
import jax.numpy as jnp
from jax.experimental import pallas as pl


def kernel(hidden_states, Wg, W1, W2):
    raise NotImplementedError("write your pallas kernel here")


# speedup vs baseline: 1.2288x; 1.2288x over previous
import functools

import jax
import jax.numpy as jnp
from jax import lax
from jax.experimental import pallas as pl
from jax.experimental.pallas import tpu as pltpu
from jax.experimental.pallas import tpu_sc as plsc

E = 8
K = 2
H = 2048
FF = 4096
T = 2048
P = T * K
BLK = 256
WMAX = 23
ROWS = WMAX * BLK
NC = 2
NS = 16
NW = NC * NS


def _router_body(h_ref, wg_ref, logits_ref, pos_ref, w16_ref, cnt_ref):
    x = h_ref[...]
    logits = jnp.dot(x, wg_ref[...], preferred_element_type=jnp.float32)
    logits_ref[...] = logits
    m = jnp.max(logits, axis=-1, keepdims=True)
    ex = jnp.exp(logits - m)
    probs = ex / jnp.sum(ex, axis=-1, keepdims=True)
    lane = lax.broadcasted_iota(jnp.int32, (T, E), 1)
    p1 = jnp.max(probs, axis=-1, keepdims=True)
    e0 = jnp.min(jnp.where(probs == p1, lane, E), axis=-1, keepdims=True)
    probs2 = jnp.where(lane == e0, -1.0, probs)
    p2 = jnp.max(probs2, axis=-1, keepdims=True)
    e1 = jnp.min(jnp.where(probs2 == p2, lane, E), axis=-1, keepdims=True)
    s = p1 + p2
    oh0 = (lane == e0).astype(jnp.float32)
    oh1 = (lane == e1).astype(jnp.float32)
    oh = jnp.concatenate([oh0, oh1], axis=0)
    cs = oh
    sh = 1
    while sh < P:
        cs = cs + jnp.concatenate(
            [jnp.zeros((sh, E), jnp.float32), cs[: P - sh]], axis=0)
        sh *= 2
    counts = cs[P - 1 : P, :]
    nblk = jnp.ceil(counts * (1.0 / BLK))
    csb = nblk
    sh = 1
    while sh < E:
        csb = csb + jnp.concatenate(
            [jnp.zeros((1, sh), jnp.float32), csb[:, : E - sh]], axis=1)
        sh *= 2
    offs = (csb - nblk) * float(BLK)
    rank = jnp.sum(oh * cs, axis=-1, keepdims=True) - 1.0
    base = jnp.sum(oh * offs, axis=-1, keepdims=True)
    pos_ref[...] = (rank + base).astype(jnp.int32)
    w = jnp.concatenate([p1 / s, p2 / s], axis=0)
    w16_ref[...] = jnp.broadcast_to(w, (P, 128))
    cnt_ref[...] = counts.astype(jnp.int32)


def _router(hf, Wg):
    return pl.pallas_call(
        _router_body,
        out_shape=(
            jax.ShapeDtypeStruct((T, E), jnp.float32),
            jax.ShapeDtypeStruct((P, 1), jnp.int32),
            jax.ShapeDtypeStruct((P, 128), jnp.float32),
            jax.ShapeDtypeStruct((1, E), jnp.int32),
        ),
    )(hf, Wg)


_PPW = P // NW
_CH = 16


def _mesh():
    return plsc.VectorSubcoreMesh(core_axis_name="c", subcore_axis_name="s",
                                  num_cores=NC, num_subcores=NS)


@functools.cache
def _make_dispatch():
    @functools.partial(
        pl.kernel,
        out_type=[
            jax.ShapeDtypeStruct((ROWS, H), jnp.float32),
            jax.ShapeDtypeStruct((ROWS, 128), jnp.float32),
        ],
        mesh=_mesh(),
        scratch_types=[
            pltpu.VMEM((_CH,), jnp.int32),
            pltpu.VMEM((_CH, H), jnp.float32),
            pltpu.VMEM((_CH, 128), jnp.float32),
            pltpu.SemaphoreType.DMA,
            pltpu.SemaphoreType.DMA,
        ],
    )
    def _dispatch(hid_hbm, w16_hbm, pos_hbm, xs_hbm, ws_hbm,
                  posbuf, xbuf, wbuf, sem1, sem2):
        wid = lax.axis_index("s") * NC + lax.axis_index("c")
        base = wid * _PPW
        for ch in range(_PPW // _CH):
            off = base + ch * _CH
            pltpu.sync_copy(pos_hbm.at[pl.ds(off, _CH)], posbuf)
            tok = (off + lax.iota(jnp.int32, _CH)) & (T - 1)
            pltpu.async_copy(hid_hbm.at[tok], xbuf, sem1).wait()
            pltpu.async_copy(xbuf, xs_hbm.at[posbuf], sem1).wait()
            pltpu.sync_copy(w16_hbm.at[pl.ds(off, _CH)], wbuf)
            pltpu.async_copy(wbuf, ws_hbm.at[posbuf], sem2).wait()

    return _dispatch


NF = 2
FT = FF // NF


def _gmm_body(we_ref, nblk_ref, x_ref, wt_ref, w1_ref, w2_ref, y_ref,
              acc_ref):
    w = pl.program_id(0)
    f = pl.program_id(1)

    @pl.when(w < nblk_ref[0])
    def _():
        x = x_ref[...].astype(jnp.bfloat16)
        h = jnp.dot(x, w1_ref[0], preferred_element_type=jnp.float32)
        h = h * jax.nn.sigmoid(h)
        part = jnp.dot(h.astype(jnp.bfloat16), w2_ref[0],
                       preferred_element_type=jnp.float32)

        @pl.when(f == 0)
        def _():
            acc_ref[...] = part

        @pl.when(f != 0)
        def _():
            acc_ref[...] = acc_ref[...] + part

        @pl.when(f == NF - 1)
        def _():
            y_ref[...] = acc_ref[...] * wt_ref[:, 0:1]


def _gmm(we, nblk, xs, ws, W1b, W2b):
    grid_spec = pltpu.PrefetchScalarGridSpec(
        num_scalar_prefetch=2,
        grid=(WMAX, NF),
        in_specs=[
            pl.BlockSpec((BLK, H), lambda w, f, we, nb: (w, 0)),
            pl.BlockSpec((BLK, 128), lambda w, f, we, nb: (w, 0)),
            pl.BlockSpec((1, H, FT), lambda w, f, we, nb: (we[w], 0, f)),
            pl.BlockSpec((1, FT, H), lambda w, f, we, nb: (we[w], f, 0)),
        ],
        out_specs=pl.BlockSpec((BLK, H), lambda w, f, we, nb: (w, 0)),
        scratch_shapes=[pltpu.VMEM((BLK, H), jnp.float32)],
    )
    return pl.pallas_call(
        _gmm_body,
        grid_spec=grid_spec,
        out_shape=jax.ShapeDtypeStruct((ROWS, H), jnp.float32),
    )(we, nblk, xs, ws, W1b, W2b)


_TPW = T // NW


@functools.cache
def _make_combine():
    @functools.partial(
        pl.kernel,
        out_type=jax.ShapeDtypeStruct((T, H), jnp.float32),
        mesh=_mesh(),
        scratch_types=[
            pltpu.VMEM((_CH,), jnp.int32),
            pltpu.VMEM((_CH,), jnp.int32),
            pltpu.VMEM((_CH, H), jnp.float32),
            pltpu.VMEM((_CH, H), jnp.float32),
            pltpu.SemaphoreType.DMA,
            pltpu.SemaphoreType.DMA,
        ],
    )
    def _combine(y_hbm, pos_hbm, out_hbm, p0buf, p1buf, y0buf, y1buf,
                 sem1, sem2):
        wid = lax.axis_index("s") * NC + lax.axis_index("c")
        base = wid * _TPW
        for ch in range(_TPW // _CH):
            t0 = base + ch * _CH
            pltpu.sync_copy(pos_hbm.at[pl.ds(t0, _CH)], p0buf)
            pltpu.sync_copy(pos_hbm.at[pl.ds(T + t0, _CH)], p1buf)
            cp0 = pltpu.async_copy(y_hbm.at[p0buf], y0buf, sem1)
            cp1 = pltpu.async_copy(y_hbm.at[p1buf], y1buf, sem2)
            cp0.wait()
            cp1.wait()
            for j in range(_CH):
                def body(i, _, j=j):
                    sl = pl.ds(i * 16, 16)
                    y0buf[j, sl] = y0buf[j, sl] + y1buf[j, sl]
                    return 0
                lax.fori_loop(0, H // 16, body, 0)
            pltpu.sync_copy(y0buf, out_hbm.at[pl.ds(t0, _CH)])

    return _combine


def kernel(hidden_states, Wg, W1, W2):
    B, S, Hd = hidden_states.shape
    hf = hidden_states.reshape(S * B, Hd)
    logits, pos2d, w16, counts = _router(hf, Wg)
    pos = pos2d.reshape(P)
    counts = counts.reshape(E)
    nblk_e = (counts + (BLK - 1)) // BLK
    blk_start = jnp.cumsum(nblk_e) - nblk_e
    warr = jnp.arange(WMAX, dtype=jnp.int32)
    we = jnp.sum((blk_start[None, :] <= warr[:, None]).astype(jnp.int32),
                 axis=1) - 1
    nblk = jnp.sum(nblk_e).reshape(1)
    xs, ws = _make_dispatch()(hf, w16, pos)
    y = _gmm(we, nblk, xs, ws,
             W1.astype(jnp.bfloat16), W2.astype(jnp.bfloat16))
    out = _make_combine()(y, pos)
    return out.reshape(B, S, Hd), logits.reshape(B, S, E)


# speedup vs baseline: 1.3596x; 1.1065x over previous
import functools

import jax
import jax.numpy as jnp
from jax import lax
from jax.experimental import pallas as pl
from jax.experimental.pallas import tpu as pltpu
from jax.experimental.pallas import tpu_sc as plsc

E = 8
K = 2
H = 2048
FF = 4096
T = 2048
P = T * K
BLK = 256
WMAX = 23
ROWS = WMAX * BLK
NC = 2
NS = 16
NW = NC * NS


def _router_body(h_ref, wg_ref, logits_ref, pos_ref, w16_ref, cnt_ref):
    x = h_ref[...]
    logits = jnp.dot(x, wg_ref[...], preferred_element_type=jnp.float32)
    logits_ref[...] = logits
    m = jnp.max(logits, axis=-1, keepdims=True)
    ex = jnp.exp(logits - m)
    probs = ex / jnp.sum(ex, axis=-1, keepdims=True)
    lane = lax.broadcasted_iota(jnp.int32, (T, E), 1)
    p1 = jnp.max(probs, axis=-1, keepdims=True)
    e0 = jnp.min(jnp.where(probs == p1, lane, E), axis=-1, keepdims=True)
    probs2 = jnp.where(lane == e0, -1.0, probs)
    p2 = jnp.max(probs2, axis=-1, keepdims=True)
    e1 = jnp.min(jnp.where(probs2 == p2, lane, E), axis=-1, keepdims=True)
    s = p1 + p2
    oh0 = (lane == e0).astype(jnp.float32)
    oh1 = (lane == e1).astype(jnp.float32)
    oh = jnp.concatenate([oh0, oh1], axis=0)
    cs = oh
    sh = 1
    while sh < P:
        cs = cs + jnp.concatenate(
            [jnp.zeros((sh, E), jnp.float32), cs[: P - sh]], axis=0)
        sh *= 2
    counts = cs[P - 1 : P, :]
    nblk = jnp.ceil(counts * (1.0 / BLK))
    csb = nblk
    sh = 1
    while sh < E:
        csb = csb + jnp.concatenate(
            [jnp.zeros((1, sh), jnp.float32), csb[:, : E - sh]], axis=1)
        sh *= 2
    offs = (csb - nblk) * float(BLK)
    rank = jnp.sum(oh * cs, axis=-1, keepdims=True) - 1.0
    base = jnp.sum(oh * offs, axis=-1, keepdims=True)
    pos_ref[...] = (rank + base).astype(jnp.int32)
    w = jnp.concatenate([p1 / s, p2 / s], axis=0)
    w16_ref[...] = jnp.broadcast_to(w, (P, 128))
    cnt_ref[...] = counts.astype(jnp.int32)


def _router(hf, Wg):
    return pl.pallas_call(
        _router_body,
        out_shape=(
            jax.ShapeDtypeStruct((T, E), jnp.float32),
            jax.ShapeDtypeStruct((P, 1), jnp.int32),
            jax.ShapeDtypeStruct((P, 128), jnp.float32),
            jax.ShapeDtypeStruct((1, E), jnp.int32),
        ),
    )(hf, Wg)


_PPW = P // NW
_CH = 16


def _mesh():
    return plsc.VectorSubcoreMesh(core_axis_name="c", subcore_axis_name="s",
                                  num_cores=NC, num_subcores=NS)


@functools.cache
def _make_dispatch():
    @functools.partial(
        pl.kernel,
        out_type=[
            jax.ShapeDtypeStruct((ROWS, H), jnp.float32),
            jax.ShapeDtypeStruct((ROWS, 128), jnp.float32),
        ],
        mesh=_mesh(),
        scratch_types=[
            pltpu.VMEM((_CH,), jnp.int32),
            pltpu.VMEM((_CH, H), jnp.float32),
            pltpu.VMEM((_CH, 128), jnp.float32),
            pltpu.SemaphoreType.DMA,
            pltpu.SemaphoreType.DMA,
        ],
    )
    def _dispatch(hid_hbm, w16_hbm, pos_hbm, xs_hbm, ws_hbm,
                  posbuf, xbuf, wbuf, sem1, sem2):
        wid = lax.axis_index("s") * NC + lax.axis_index("c")
        base = wid * _PPW
        for ch in range(_PPW // _CH):
            off = base + ch * _CH
            pltpu.sync_copy(pos_hbm.at[pl.ds(off, _CH)], posbuf)
            tok = (off + lax.iota(jnp.int32, _CH)) & (T - 1)
            pltpu.async_copy(hid_hbm.at[tok], xbuf, sem1).wait()
            pltpu.async_copy(xbuf, xs_hbm.at[posbuf], sem1).wait()
            pltpu.sync_copy(w16_hbm.at[pl.ds(off, _CH)], wbuf)
            pltpu.async_copy(wbuf, ws_hbm.at[posbuf], sem2).wait()

    return _dispatch


NF = 4
FT = FF // NF


def _gmm_body(we_ref, nblk_ref, x_ref, wt_ref, w1_ref, w2_ref, yin_ref,
              y_ref):
    f = pl.program_id(0)
    w = pl.program_id(1)

    @pl.when(w < nblk_ref[0])
    def _():
        x = x_ref[...]
        h = jnp.dot(x, w1_ref[0], preferred_element_type=jnp.float32)
        h = h * jax.nn.sigmoid(h)
        part = jnp.dot(h, w2_ref[0], preferred_element_type=jnp.float32)

        @pl.when(f == 0)
        def _():
            y_ref[...] = part

        @pl.when(f == NF - 1)
        def _():
            y_ref[...] = (yin_ref[...] + part) * wt_ref[:, 0:1]

        @pl.when((f != 0) & (f != NF - 1))
        def _():
            y_ref[...] = yin_ref[...] + part


def _gmm(we, nblk, xs, ws, W1, W2):
    grid_spec = pltpu.PrefetchScalarGridSpec(
        num_scalar_prefetch=2,
        grid=(NF, WMAX),
        in_specs=[
            pl.BlockSpec((BLK, H), lambda f, w, we, nb: (w, 0)),
            pl.BlockSpec((BLK, 128), lambda f, w, we, nb: (w, 0)),
            pl.BlockSpec((1, H, FT), lambda f, w, we, nb: (we[w], 0, f)),
            pl.BlockSpec((1, FT, H), lambda f, w, we, nb: (we[w], f, 0)),
            pl.BlockSpec((BLK, H), lambda f, w, we, nb: (w, 0)),
        ],
        out_specs=pl.BlockSpec((BLK, H), lambda f, w, we, nb: (w, 0)),
    )
    yin = jnp.zeros((ROWS, H), jnp.float32)
    return pl.pallas_call(
        _gmm_body,
        grid_spec=grid_spec,
        out_shape=jax.ShapeDtypeStruct((ROWS, H), jnp.float32),
        input_output_aliases={6: 0},
    )(we, nblk, xs, ws, W1, W2, yin)


_TPW = T // NW


@functools.cache
def _make_combine():
    @functools.partial(
        pl.kernel,
        out_type=jax.ShapeDtypeStruct((T, H), jnp.float32),
        mesh=_mesh(),
        scratch_types=[
            pltpu.VMEM((_CH,), jnp.int32),
            pltpu.VMEM((_CH,), jnp.int32),
            pltpu.VMEM((_CH, H), jnp.float32),
            pltpu.VMEM((_CH, H), jnp.float32),
            pltpu.SemaphoreType.DMA,
            pltpu.SemaphoreType.DMA,
        ],
    )
    def _combine(y_hbm, pos_hbm, out_hbm, p0buf, p1buf, y0buf, y1buf,
                 sem1, sem2):
        wid = lax.axis_index("s") * NC + lax.axis_index("c")
        base = wid * _TPW
        for ch in range(_TPW // _CH):
            t0 = base + ch * _CH
            pltpu.sync_copy(pos_hbm.at[pl.ds(t0, _CH)], p0buf)
            pltpu.sync_copy(pos_hbm.at[pl.ds(T + t0, _CH)], p1buf)
            cp0 = pltpu.async_copy(y_hbm.at[p0buf], y0buf, sem1)
            cp1 = pltpu.async_copy(y_hbm.at[p1buf], y1buf, sem2)
            cp0.wait()
            cp1.wait()
            for j in range(_CH):
                def body(i, _, j=j):
                    sl = pl.ds(i * 16, 16)
                    y0buf[j, sl] = y0buf[j, sl] + y1buf[j, sl]
                    return 0
                lax.fori_loop(0, H // 16, body, 0)
            pltpu.sync_copy(y0buf, out_hbm.at[pl.ds(t0, _CH)])

    return _combine


def kernel(hidden_states, Wg, W1, W2):
    B, S, Hd = hidden_states.shape
    hf = hidden_states.reshape(S * B, Hd)
    logits, pos2d, w16, counts = _router(hf, Wg)
    pos = pos2d.reshape(P)
    counts = counts.reshape(E)
    nblk_e = (counts + (BLK - 1)) // BLK
    blk_start = jnp.cumsum(nblk_e) - nblk_e
    warr = jnp.arange(WMAX, dtype=jnp.int32)
    we = jnp.sum((blk_start[None, :] <= warr[:, None]).astype(jnp.int32),
                 axis=1) - 1
    nblk = jnp.sum(nblk_e).reshape(1)
    xs, ws = _make_dispatch()(hf, w16, pos)
    y = _gmm(we, nblk, xs, ws, W1, W2)
    out = _make_combine()(y, pos)
    return out.reshape(B, S, Hd), logits.reshape(B, S, E)


# speedup vs baseline: 1.9925x; 1.4655x over previous
import functools

import jax
import jax.numpy as jnp
from jax import lax
from jax.experimental import pallas as pl
from jax.experimental.pallas import tpu as pltpu
from jax.experimental.pallas import tpu_sc as plsc

E = 8
K = 2
H = 2048
FF = 4096
T = 2048
P = T * K
BLK = 256
WMAX = 23
ROWS = WMAX * BLK
NC = 2
NS = 16
NW = NC * NS


def _router_body(h_ref, wg_ref, logits_ref, pos_ref, w16_ref, cnt_ref):
    x = h_ref[...]
    logits = jnp.dot(x, wg_ref[...], preferred_element_type=jnp.float32)
    logits_ref[...] = logits
    m = jnp.max(logits, axis=-1, keepdims=True)
    ex = jnp.exp(logits - m)
    probs = ex / jnp.sum(ex, axis=-1, keepdims=True)
    lane = lax.broadcasted_iota(jnp.int32, (T, E), 1)
    p1 = jnp.max(probs, axis=-1, keepdims=True)
    e0 = jnp.min(jnp.where(probs == p1, lane, E), axis=-1, keepdims=True)
    probs2 = jnp.where(lane == e0, -1.0, probs)
    p2 = jnp.max(probs2, axis=-1, keepdims=True)
    e1 = jnp.min(jnp.where(probs2 == p2, lane, E), axis=-1, keepdims=True)
    s = p1 + p2
    oh0 = (lane == e0).astype(jnp.float32)
    oh1 = (lane == e1).astype(jnp.float32)
    oh = jnp.concatenate([oh0, oh1], axis=0)
    cs = oh
    sh = 1
    while sh < P:
        cs = cs + jnp.concatenate(
            [jnp.zeros((sh, E), jnp.float32), cs[: P - sh]], axis=0)
        sh *= 2
    counts = cs[P - 1 : P, :]
    nblk = jnp.ceil(counts * (1.0 / BLK))
    csb = nblk
    sh = 1
    while sh < E:
        csb = csb + jnp.concatenate(
            [jnp.zeros((1, sh), jnp.float32), csb[:, : E - sh]], axis=1)
        sh *= 2
    offs = (csb - nblk) * float(BLK)
    rank = jnp.sum(oh * cs, axis=-1, keepdims=True) - 1.0
    base = jnp.sum(oh * offs, axis=-1, keepdims=True)
    pos_ref[...] = (rank + base).astype(jnp.int32)
    w = jnp.concatenate([p1 / s, p2 / s], axis=0)
    w16_ref[...] = jnp.broadcast_to(w, (P, 128))
    cnt_ref[...] = counts.astype(jnp.int32)


def _router(hf, Wg):
    return pl.pallas_call(
        _router_body,
        out_shape=(
            jax.ShapeDtypeStruct((T, E), jnp.float32),
            jax.ShapeDtypeStruct((P, 1), jnp.int32),
            jax.ShapeDtypeStruct((P, 128), jnp.float32),
            jax.ShapeDtypeStruct((1, E), jnp.int32),
        ),
    )(hf, Wg)


_PPW = P // NW
_CH = 16


def _mesh():
    return plsc.VectorSubcoreMesh(core_axis_name="c", subcore_axis_name="s",
                                  num_cores=NC, num_subcores=NS)


_NBUF = 3


@functools.cache
def _make_dispatch():
    @functools.partial(
        pl.kernel,
        out_type=[
            jax.ShapeDtypeStruct((ROWS, H), jnp.float32),
            jax.ShapeDtypeStruct((ROWS, 128), jnp.float32),
        ],
        mesh=_mesh(),
        scratch_types=(
            [pltpu.VMEM((_PPW,), jnp.int32),
             pltpu.VMEM((_PPW, 128), jnp.float32)]
            + [pltpu.VMEM((_CH, H), jnp.float32)] * _NBUF
            + [pltpu.SemaphoreType.DMA] * (2 * _NBUF + 2)
        ),
    )
    def _dispatch(hid_hbm, w16_hbm, pos_hbm, xs_hbm, ws_hbm, *scr):
        posall, wall = scr[0], scr[1]
        xb = scr[2:2 + _NBUF]
        gsem = scr[2 + _NBUF:2 + 2 * _NBUF]
        wsem = scr[2 + 2 * _NBUF:4 + 2 * _NBUF]
        ssem = scr[4 + 2 * _NBUF:4 + 3 * _NBUF]
        wid = lax.axis_index("s") * NC + lax.axis_index("c")
        base = wid * _PPW
        nch = _PPW // _CH
        pltpu.sync_copy(pos_hbm.at[pl.ds(base, _PPW)], posall)
        wcp = pltpu.async_copy(w16_hbm.at[pl.ds(base, _PPW)], wall, wsem[0])
        g = [None] * nch
        sc = [None] * nch
        sw = [None] * nch
        for c in range(nch + 1):
            if c < nch:
                b = c % _NBUF
                if c >= _NBUF:
                    sc[c - _NBUF].wait()
                off = base + c * _CH
                tok = (off + lax.iota(jnp.int32, _CH)) & (T - 1)
                g[c] = pltpu.async_copy(hid_hbm.at[tok], xb[b], gsem[b])
            if c == 1:
                wcp.wait()
            if c >= 1:
                bb = (c - 1) % _NBUF
                pv = posall[pl.ds((c - 1) * _CH, _CH)]
                g[c - 1].wait()
                sc[c - 1] = pltpu.async_copy(xb[bb], xs_hbm.at[pv], ssem[bb])
                sw[c - 1] = pltpu.async_copy(
                    wall.at[pl.ds((c - 1) * _CH, _CH)], ws_hbm.at[pv],
                    wsem[(c - 1) % 2])
        for c in range(max(0, nch - _NBUF), nch):
            sc[c].wait()
        for c in range(nch):
            sw[c].wait()

    return _dispatch


NT = 16
TS = FF // NT


def _gmm_body(we_ref, nblk_ref, x_ref, wt_ref, w1_hbm, w2_hbm, y_ref,
              w1c, w2c, st1, st2, sems):
    w = pl.program_id(0)
    e = we_ref[w]
    prev = we_ref[jnp.maximum(w - 1, 0)]
    valid = w < nblk_ref[0]
    changed = ((w == 0) | (e != prev)) & valid

    def tile_copies(t):
        s = t % 3
        c1 = pltpu.make_async_copy(
            w1_hbm.at[e, :, pl.ds(t * TS, TS)], st1.at[s], sems.at[s])
        c2 = pltpu.make_async_copy(
            w2_hbm.at[e, pl.ds(t * TS, TS), :], st2.at[s], sems.at[3 + s])
        return c1, c2

    @pl.when(changed)
    def _():
        x = x_ref[...].astype(jnp.bfloat16)
        acc = jnp.zeros((BLK, H), jnp.float32)
        for t0 in range(2):
            c1, c2 = tile_copies(t0)
            c1.start()
            c2.start()
        for t in range(NT):
            s = t % 3
            if t + 2 < NT:
                n1, n2 = tile_copies(t + 2)
                n1.start()
                n2.start()
            c1, c2 = tile_copies(t)
            c1.wait()
            c2.wait()
            w1t = st1[s].astype(jnp.bfloat16)
            w2t = st2[s].astype(jnp.bfloat16)
            w1c[:, pl.ds(t * TS, TS)] = w1t
            w2c[pl.ds(t * TS, TS), :] = w2t
            ht = jnp.dot(x, w1t, preferred_element_type=jnp.float32)
            ht = ht * jax.nn.sigmoid(ht)
            acc = acc + jnp.dot(ht.astype(jnp.bfloat16), w2t,
                                preferred_element_type=jnp.float32)
        y_ref[...] = acc * wt_ref[:, 0:1]

    @pl.when(jnp.logical_not(changed) & valid)
    def _():
        x = x_ref[...].astype(jnp.bfloat16)
        h = jnp.dot(x, w1c[...], preferred_element_type=jnp.float32)
        h = h * jax.nn.sigmoid(h)
        y = jnp.dot(h.astype(jnp.bfloat16), w2c[...],
                    preferred_element_type=jnp.float32)
        y_ref[...] = y * wt_ref[:, 0:1]


def _gmm(we, nblk, xs, ws, W1, W2):
    grid_spec = pltpu.PrefetchScalarGridSpec(
        num_scalar_prefetch=2,
        grid=(WMAX,),
        in_specs=[
            pl.BlockSpec((BLK, H), lambda w, we, nb: (w, 0)),
            pl.BlockSpec((BLK, 128), lambda w, we, nb: (w, 0)),
            pl.BlockSpec(memory_space=pltpu.MemorySpace.HBM),
            pl.BlockSpec(memory_space=pltpu.MemorySpace.HBM),
        ],
        out_specs=pl.BlockSpec((BLK, H), lambda w, we, nb: (w, 0)),
        scratch_shapes=[
            pltpu.VMEM((H, FF), jnp.bfloat16),
            pltpu.VMEM((FF, H), jnp.bfloat16),
            pltpu.VMEM((3, H, TS), jnp.float32),
            pltpu.VMEM((3, TS, H), jnp.float32),
            pltpu.SemaphoreType.DMA((6,)),
        ],
    )
    return pl.pallas_call(
        _gmm_body,
        grid_spec=grid_spec,
        out_shape=jax.ShapeDtypeStruct((ROWS, H), jnp.float32),
    )(we, nblk, xs, ws, W1, W2)


_TPW = T // NW


_CCH = 8


@functools.cache
def _make_combine():
    @functools.partial(
        pl.kernel,
        out_type=jax.ShapeDtypeStruct((T, H), jnp.float32),
        mesh=_mesh(),
        scratch_types=(
            [pltpu.VMEM((_TPW,), jnp.int32)] * 2
            + [pltpu.VMEM((_CCH, H), jnp.float32)] * 4
            + [pltpu.SemaphoreType.DMA] * 6
        ),
    )
    def _combine(y_hbm, pos_hbm, out_hbm, *scr):
        p0all, p1all = scr[0], scr[1]
        y0b, y1b = scr[2:4], scr[4:6]
        g0s, g1s, sts = scr[6:8], scr[8:10], scr[10:12]
        wid = lax.axis_index("s") * NC + lax.axis_index("c")
        base = wid * _TPW
        nch = _TPW // _CCH
        pltpu.sync_copy(pos_hbm.at[pl.ds(base, _TPW)], p0all)
        pltpu.sync_copy(pos_hbm.at[pl.ds(T + base, _TPW)], p1all)
        g0 = [None] * nch
        g1 = [None] * nch
        st = [None] * nch

        def add_store(c):
            b = c % 2
            g0[c].wait()
            g1[c].wait()
            for j in range(_CCH):
                @plsc.parallel_loop(0, H // 16, 1, unroll=8)
                def _(i, j=j, b=b):
                    sl = pl.ds(i * 16, 16)
                    y0b[b][j, sl] = y0b[b][j, sl] + y1b[b][j, sl]
            st[c] = pltpu.async_copy(
                y0b[b], out_hbm.at[pl.ds(base + c * _CCH, _CCH)], sts[b])

        for c in range(nch + 1):
            if c < nch:
                b = c % 2
                if c >= 2:
                    st[c - 2].wait()
                p0 = p0all.at[pl.ds(c * _CCH, _CCH)]
                p1 = p1all.at[pl.ds(c * _CCH, _CCH)]
                g0[c] = pltpu.async_copy(y_hbm.at[p0], y0b[b], g0s[b])
                g1[c] = pltpu.async_copy(y_hbm.at[p1], y1b[b], g1s[b])
            if c >= 1:
                add_store(c - 1)
        st[nch - 2].wait()
        st[nch - 1].wait()

    return _combine


def kernel(hidden_states, Wg, W1, W2):
    B, S, Hd = hidden_states.shape
    hf = hidden_states.reshape(S * B, Hd)
    logits, pos2d, w16, counts = _router(hf, Wg)
    pos = pos2d.reshape(P)
    counts = counts.reshape(E)
    nblk_e = (counts + (BLK - 1)) // BLK
    blk_start = jnp.cumsum(nblk_e) - nblk_e
    warr = jnp.arange(WMAX, dtype=jnp.int32)
    we = jnp.sum((blk_start[None, :] <= warr[:, None]).astype(jnp.int32),
                 axis=1) - 1
    nblk = jnp.sum(nblk_e).reshape(1)
    xs, ws = _make_dispatch()(hf, w16, pos)
    y = _gmm(we, nblk, xs, ws, W1, W2)
    out = _make_combine()(y, pos)
    return out.reshape(B, S, Hd), logits.reshape(B, S, E)
